# baseline (speedup 1.0000x reference)
import functools

import jax
import jax.numpy as jnp
from jax import lax
from jax.experimental import pallas as pl
from jax.experimental.pallas import tpu as pltpu
from jax.experimental.pallas import tpu_sc as plsc

NENTITY = 1000000
NRELATION = 1000000
HIDDEN = 64
BATCH = 16384

NC = 2
NS = 16
NW = NC * NS
BPW = BATCH // NW
NCHUNK = BPW // 128


def _body(h_idx_hbm, r_idx_hbm, t_idx_hbm, gamma_hbm,
          entity_hbm, relation_hbm, out_hbm,
          idx_h, idx_r, idx_t, rows_h, rows_r, rows_t,
          gamma_v, out_v, sem):
    wid = lax.axis_index("s") * NC + lax.axis_index("c")
    base = wid * BPW

    pltpu.sync_copy(h_idx_hbm.at[wid], idx_h)
    pltpu.sync_copy(r_idx_hbm.at[wid], idx_r)
    pltpu.sync_copy(t_idx_hbm.at[wid], idx_t)
    pltpu.sync_copy(gamma_hbm, gamma_v)

    copies = []
    for j in range(NCHUNK):
        dst = pl.ds(j * 128, 128)
        copies.append(pltpu.async_copy(entity_hbm.at[idx_h.at[j]],
                                       rows_h.at[dst], sem))
        copies.append(pltpu.async_copy(relation_hbm.at[idx_r.at[j]],
                                       rows_r.at[dst], sem))
        copies.append(pltpu.async_copy(entity_hbm.at[idx_t.at[j]],
                                       rows_t.at[dst], sem))
    for c in copies:
        c.wait()

    gamma_vec = gamma_v[...]
    lane = lax.iota(jnp.int32, 16)

    def group(g, carry):
        score = gamma_vec
        for i in range(16):
            row = g * 16 + i
            acc = None
            for c in range(HIDDEN // 16):
                d = pl.ds(c * 16, 16)
                v = jnp.abs(rows_h[row, d] + rows_r[row, d] - rows_t[row, d])
                acc = v if acc is None else acc + v
            s = jnp.sum(acc)
            score = jnp.where(lane == i, score - s, score)
        out_v[pl.ds(g * 16, 16)] = score
        return carry

    lax.fori_loop(0, BPW // 16, group, 0)

    pltpu.sync_copy(out_v, out_hbm.at[pl.ds(base, BPW)])


@jax.jit
def _transe_scores(h_idx, r_idx, t_idx, gamma16, entity_embedding,
                   relation_embedding):
    mesh = plsc.VectorSubcoreMesh(core_axis_name="c", subcore_axis_name="s")
    f = pl.kernel(
        _body, mesh=mesh,
        compiler_params=pltpu.CompilerParams(
            needs_layout_passes=False, use_tc_tiling_on_sc=False),
        out_type=jax.ShapeDtypeStruct((BATCH,), jnp.float32),
        scratch_types=[
            pltpu.VMEM((NCHUNK, 128), jnp.int32),
            pltpu.VMEM((NCHUNK, 128), jnp.int32),
            pltpu.VMEM((NCHUNK, 128), jnp.int32),
            pltpu.VMEM((BPW, HIDDEN), jnp.float32),
            pltpu.VMEM((BPW, HIDDEN), jnp.float32),
            pltpu.VMEM((BPW, HIDDEN), jnp.float32),
            pltpu.VMEM((16,), jnp.float32),
            pltpu.VMEM((BPW,), jnp.float32),
            pltpu.SemaphoreType.DMA,
        ],
    )
    return f(h_idx, r_idx, t_idx, gamma16, entity_embedding,
             relation_embedding)


def kernel(sample, entity_embedding, relation_embedding, gamma):
    sample = sample.astype(jnp.int32)
    h_idx = sample[:, 0].reshape(NW, NCHUNK, 128)
    r_idx = sample[:, 1].reshape(NW, NCHUNK, 128)
    t_idx = sample[:, 2].reshape(NW, NCHUNK, 128)
    gamma16 = jnp.broadcast_to(gamma.astype(jnp.float32), (16,))
    scores = _transe_scores(h_idx, r_idx, t_idx, gamma16,
                            entity_embedding, relation_embedding)
    return scores.reshape(BATCH, 1)
